# Initial kernel scaffold; baseline (speedup 1.0000x reference)
import jax
import jax.numpy as jnp
from jax.experimental import pallas as pl


def kernel(token_embeds, cls_token_ids, ln_gamma, ln_beta, W, b):
    raise NotImplementedError("write your pallas kernel here")


# speedup vs baseline: 2.3254x; 2.3254x over previous
import functools

import jax
import jax.numpy as jnp
from jax import lax
from jax.experimental import pallas as pl
from jax.experimental.pallas import tpu as pltpu
from jax.experimental.pallas import tpu_sc as plsc

_NUM_CORES = 2
_NUM_SUBCORES = 16
_NUM_WORKERS = _NUM_CORES * _NUM_SUBCORES


def _sc_gather(table, idx_flat, rows_per_batch, seq_len):
    total_rows, hidden = idx_flat.shape[0], table.shape[1]
    rpw = total_rows // _NUM_WORKERS

    mesh = plsc.VectorSubcoreMesh(core_axis_name="c", subcore_axis_name="s")

    @functools.partial(
        pl.kernel,
        out_type=jax.ShapeDtypeStruct((total_rows, hidden), jnp.float32),
        mesh=mesh,
        scratch_types=[
            pltpu.VMEM((rpw,), jnp.int32),
            pltpu.VMEM((rpw, hidden), jnp.float32),
            pltpu.SemaphoreType.DMA,
        ],
    )
    def gather_kernel(table_hbm, idx_hbm, out_hbm, idx_v, rows_v, sem):
        wid = lax.axis_index("s") * _NUM_CORES + lax.axis_index("c")
        base = wid * rpw
        pltpu.sync_copy(idx_hbm.at[pl.ds(base, rpw)], idx_v)
        row_off = (base // rows_per_batch) * seq_len
        idx_v[...] = idx_v[...] + row_off
        pltpu.async_copy(table_hbm.at[idx_v], rows_v, sem).wait()
        pltpu.sync_copy(rows_v, out_hbm.at[pl.ds(base, rpw)])

    return gather_kernel(table, idx_flat)


def _head_body(cls_ref, ids_ref, g_ref, bta_ref, w_ref, bb_ref,
               logits_ref, mask_ref):
    x = cls_ref[...]
    hidden = x.shape[1]
    mean = jnp.mean(x, axis=1, keepdims=True)
    xc = x - mean
    var = jnp.mean(xc * xc, axis=1, keepdims=True)
    inv = lax.rsqrt(var + 1e-6)
    normed = xc * inv * g_ref[...] + bta_ref[...]
    logit = jnp.sum(normed * w_ref[...], axis=1)
    logits_ref[...] = logit.reshape(logits_ref.shape) + bb_ref[...]
    mask_ref[...] = (ids_ref[...] != -1).astype(jnp.float32)


def _tc_head(cls_flat, cls_token_ids, ln_gamma, ln_beta, w_row, b):
    bsz, seq = cls_token_ids.shape
    logits, mask = pl.pallas_call(
        _head_body,
        out_shape=[
            jax.ShapeDtypeStruct((bsz, seq), jnp.float32),
            jax.ShapeDtypeStruct((bsz, seq), jnp.float32),
        ],
    )(cls_flat, cls_token_ids, ln_gamma.reshape(1, -1),
      ln_beta.reshape(1, -1), w_row, b.reshape(1, 1))
    return logits, mask


def kernel(token_embeds, cls_token_ids, ln_gamma, ln_beta, W, b):
    bsz, seq_len, hidden = token_embeds.shape
    s = cls_token_ids.shape[1]
    table = token_embeds.reshape(bsz * seq_len, hidden)
    idx_flat = cls_token_ids.reshape(-1).astype(jnp.int32)

    cls_flat = _sc_gather(table, idx_flat, s, seq_len)
    logits, mask = _tc_head(cls_flat, cls_token_ids, ln_gamma, ln_beta,
                            W.reshape(1, hidden), b)
    cls_vec = cls_flat.reshape(bsz, s, hidden)
    return (logits, cls_vec, mask)
